# baseline (speedup 1.0000x reference)
import functools

import jax
import jax.numpy as jnp
from jax import lax
from jax.experimental import pallas as pl
from jax.experimental.pallas import tpu as pltpu
from jax.experimental.pallas import tpu_sc as plsc

N_NODES = 10000
N_EDGES = 320000
D = 128
DEG = 32

BN = 4
BE = BN * DEG
NBLOCKS = N_NODES // BN
NW = 32
NCHUNK = D // 32
NBUF = 4
KMAX = 80
_HI = -65536


def _body(x_hbm, w_hbm, idx_hbm, out_hbm,
          idx_v, rows0, rows1, rows2, rows3, w0, w1, w2, w3,
          out0, out1, out2, out3,
          sg0, sg1, sg2, sg3, sw0, sw1, sw2, sw3, so0, so1, so2, so3):
    rows = (rows0, rows1, rows2, rows3)
    wv = (w0, w1, w2, w3)
    outv = (out0, out1, out2, out3)
    sg = (sg0, sg1, sg2, sg3)
    sw = (sw0, sw1, sw2, sw3)
    so = (so0, so1, so2, so3)

    wid = lax.axis_index("s") * 2 + lax.axis_index("c")
    start = (wid * NBLOCKS) // NW
    end = ((wid + 1) * NBLOCKS) // NW
    count = end - start

    pltpu.sync_copy(idx_hbm.at[pl.ds(start * BE, 78 * BE)], idx_v.at[pl.ds(0, 78 * BE)])

    @pl.when(count > 78)
    def _():
        pltpu.sync_copy(idx_hbm.at[pl.ds((start + 78) * BE, BE)],
                        idx_v.at[pl.ds(78 * BE, BE)])

    def kth_block(k):
        kk = jnp.minimum(k, count - 1)
        return kk, start + kk

    def issue(k, b):
        kk, block = kth_block(k)
        pltpu.async_copy(x_hbm.at[idx_v.at[pl.ds(kk * BE, BE)]], rows[b], sg[b])
        pltpu.async_copy(w_hbm.at[pl.ds(block * BE, BE)], wv[b], sw[b])

    for b in range(NBUF):
        issue(b, b)

    def step(t, _):
        for b in range(NBUF):
            k = NBUF * t + b
            kk, block = kth_block(k)
            pltpu.make_async_copy(
                x_hbm.at[idx_v.at[pl.ds(kk * BE, BE)]], rows[b], sg[b]).wait()
            pltpu.make_async_copy(
                w_hbm.at[pl.ds(block * BE, BE)], wv[b], sw[b]).wait()

            @pl.when(t >= 1)
            def _():
                pltpu.make_async_copy(
                    outv[b], out_hbm.at[pl.ds(block * BN, BN)], so[b]).wait()

            for n in range(BN):
                def jbody(j4, acc, n=n, b=b):
                    res = list(acc)
                    for u in range(4):
                        e = n * DEG + j4 * 4 + u
                        for c in range(NCHUNK):
                            ri = rows[b][e, pl.ds(c * 16, 16)]
                            ra = lax.bitcast_convert_type(ri << 16, jnp.float32)
                            rb = lax.bitcast_convert_type(ri & _HI, jnp.float32)
                            res[2 * c] = res[2 * c] + ra * wv[b][
                                e, pl.ds(c * 32, 16)]
                            res[2 * c + 1] = res[2 * c + 1] + rb * wv[b][
                                e, pl.ds(c * 32 + 16, 16)]
                    return tuple(res)

                acc = lax.fori_loop(
                    0, DEG // 4, jbody,
                    tuple(jnp.zeros((16,), jnp.float32) for _ in range(2 * NCHUNK)),
                )
                for h in range(2 * NCHUNK):
                    outv[b][n, pl.ds(h * 16, 16)] = acc[h] * (1.0 / DEG)

            pltpu.async_copy(outv[b], out_hbm.at[pl.ds(block * BN, BN)], so[b])

            @pl.when(t < KMAX // NBUF - 1)
            def _():
                issue(k + NBUF, b)
        return 0

    lax.fori_loop(0, KMAX // NBUF, step, 0)

    for b in range(NBUF):
        pltpu.make_async_copy(outv[b], out_hbm.at[pl.ds(0, BN)], so[b]).wait()


@jax.jit
def _graph_conv(x, w, idx):
    xp = x.reshape(N_NODES, NCHUNK, 2, 16).transpose(0, 1, 3, 2)
    xp = xp.reshape(N_NODES, D // 2, 2).astype(jnp.bfloat16)
    xp = jax.lax.bitcast_convert_type(xp, jnp.int32)
    mesh = plsc.VectorSubcoreMesh(core_axis_name="c", subcore_axis_name="s")
    k = functools.partial(
        pl.kernel,
        mesh=mesh,
        compiler_params=pltpu.CompilerParams(use_tc_tiling_on_sc=False),
        out_type=jax.ShapeDtypeStruct((N_NODES, D), jnp.float32),
        scratch_types=(
            [pltpu.VMEM((79 * BE,), jnp.int32)]
            + [pltpu.VMEM((BE, D // 2), jnp.int32)] * NBUF
            + [pltpu.VMEM((BE, D), jnp.float32)] * NBUF
            + [pltpu.VMEM((BN, D), jnp.float32)] * NBUF
            + [pltpu.SemaphoreType.DMA] * (3 * NBUF)
        ),
    )(_body)
    return k(xp, w, idx)


def kernel(input, weights, idxn):
    return _graph_conv(input, weights, idxn)


# speedup vs baseline: 1.0568x; 1.0568x over previous
import functools

import jax
import jax.numpy as jnp
from jax import lax
from jax.experimental import pallas as pl
from jax.experimental.pallas import tpu as pltpu
from jax.experimental.pallas import tpu_sc as plsc

N_NODES = 10000
N_EDGES = 320000
D = 128
DEG = 32

BN = 4
BE = BN * DEG
NBLOCKS = N_NODES // BN
NW = 32
NCHUNK = D // 32
NBUF = 3
KMAX = 81
_HI = -65536


def _body(x_hbm, w_hbm, idx_hbm, out_hbm,
          idx_v, rows0, rows1, rows2, w0, w1, w2, out0, out1, out2,
          sg0, sg1, sg2, sw0, sw1, sw2, so0, so1, so2):
    rows = (rows0, rows1, rows2)
    wv = (w0, w1, w2)
    outv = (out0, out1, out2)
    sg = (sg0, sg1, sg2)
    sw = (sw0, sw1, sw2)
    so = (so0, so1, so2)

    wid = lax.axis_index("s") * 2 + lax.axis_index("c")
    start = (wid * NBLOCKS) // NW
    end = ((wid + 1) * NBLOCKS) // NW
    count = end - start

    pltpu.sync_copy(idx_hbm.at[pl.ds(start * BE, 78 * BE)], idx_v.at[pl.ds(0, 78 * BE)])

    @pl.when(count > 78)
    def _():
        pltpu.sync_copy(idx_hbm.at[pl.ds((start + 78) * BE, BE)],
                        idx_v.at[pl.ds(78 * BE, BE)])

    def kth_block(k):
        kk = jnp.minimum(k, count - 1)
        return kk, start + kk

    def issue(k, b):
        kk, block = kth_block(k)
        pltpu.async_copy(x_hbm.at[idx_v.at[pl.ds(kk * BE, BE)]], rows[b], sg[b])
        pltpu.async_copy(w_hbm.at[pl.ds(block * BE, BE)], wv[b], sw[b])

    for b in range(NBUF):
        issue(b, b)

    def step(t, _):
        for b in range(NBUF):
            k = NBUF * t + b
            kk, block = kth_block(k)
            pltpu.make_async_copy(
                x_hbm.at[idx_v.at[pl.ds(kk * BE, BE)]], rows[b], sg[b]).wait()
            pltpu.make_async_copy(
                w_hbm.at[pl.ds(block * BE, BE)], wv[b], sw[b]).wait()

            @pl.when(t >= 1)
            def _():
                pltpu.make_async_copy(
                    outv[b], out_hbm.at[pl.ds(block * BN, BN)], so[b]).wait()

            for n in range(BN):
                def jbody(j4, acc, n=n, b=b):
                    res = list(acc)
                    for u in range(4):
                        e = n * DEG + j4 * 4 + u
                        for c in range(NCHUNK):
                            ri = rows[b][e, pl.ds(c * 16, 16)]
                            ra = lax.bitcast_convert_type(ri << 16, jnp.float32)
                            rb = lax.bitcast_convert_type(ri & _HI, jnp.float32)
                            res[2 * c] = res[2 * c] + ra * wv[b][
                                e, pl.ds(c * 32, 16)]
                            res[2 * c + 1] = res[2 * c + 1] + rb * wv[b][
                                e, pl.ds(c * 32 + 16, 16)]
                    return tuple(res)

                acc = lax.fori_loop(
                    0, DEG // 4, jbody,
                    tuple(jnp.zeros((16,), jnp.float32) for _ in range(2 * NCHUNK)),
                )
                for h in range(2 * NCHUNK):
                    outv[b][n, pl.ds(h * 16, 16)] = acc[h] * (1.0 / DEG)

            pltpu.async_copy(outv[b], out_hbm.at[pl.ds(block * BN, BN)], so[b])

            @pl.when(t < KMAX // NBUF - 1)
            def _():
                issue(k + NBUF, b)
        return 0

    lax.fori_loop(0, KMAX // NBUF, step, 0)

    for b in range(NBUF):
        pltpu.make_async_copy(outv[b], out_hbm.at[pl.ds(0, BN)], so[b]).wait()


@jax.jit
def _graph_conv(x, w, idx):
    xp = x.reshape(N_NODES, NCHUNK, 2, 16).transpose(0, 1, 3, 2)
    xp = xp.reshape(N_NODES, D // 2, 2).astype(jnp.bfloat16)
    xp = jax.lax.bitcast_convert_type(xp, jnp.int32)
    mesh = plsc.VectorSubcoreMesh(core_axis_name="c", subcore_axis_name="s")
    k = functools.partial(
        pl.kernel,
        mesh=mesh,
        compiler_params=pltpu.CompilerParams(use_tc_tiling_on_sc=False),
        out_type=jax.ShapeDtypeStruct((N_NODES, D), jnp.float32),
        scratch_types=[
            pltpu.VMEM((79 * BE,), jnp.int32),
            pltpu.VMEM((BE, D // 2), jnp.int32),
            pltpu.VMEM((BE, D // 2), jnp.int32),
            pltpu.VMEM((BE, D // 2), jnp.int32),
            pltpu.VMEM((BE, D), jnp.float32),
            pltpu.VMEM((BE, D), jnp.float32),
            pltpu.VMEM((BE, D), jnp.float32),
            pltpu.VMEM((BN, D), jnp.float32),
            pltpu.VMEM((BN, D), jnp.float32),
            pltpu.VMEM((BN, D), jnp.float32),
            pltpu.SemaphoreType.DMA,
            pltpu.SemaphoreType.DMA,
            pltpu.SemaphoreType.DMA,
            pltpu.SemaphoreType.DMA,
            pltpu.SemaphoreType.DMA,
            pltpu.SemaphoreType.DMA,
            pltpu.SemaphoreType.DMA,
            pltpu.SemaphoreType.DMA,
            pltpu.SemaphoreType.DMA,
        ],
    )(_body)
    return k(xp, w, idx)


def kernel(input, weights, idxn):
    return _graph_conv(input, weights, idxn)


# speedup vs baseline: 1.0595x; 1.0025x over previous
import functools

import jax
import jax.numpy as jnp
from jax import lax
from jax.experimental import pallas as pl
from jax.experimental.pallas import tpu as pltpu
from jax.experimental.pallas import tpu_sc as plsc

N_NODES = 10000
N_EDGES = 320000
D = 128
DEG = 32

BN = 4
BE = BN * DEG
NBLOCKS = N_NODES // BN
NW = 32
NCHUNK = D // 32
NBUF = 3
KMAX = 81
_HI = -65536


def _body(x_hbm, w_hbm, idx_hbm, out_hbm,
          idx_v, rows0, rows1, rows2, w0, w1, w2, out0, out1, out2,
          sg0, sg1, sg2, sw0, sw1, sw2, so0, so1, so2):
    rows = (rows0, rows1, rows2)
    wv = (w0, w1, w2)
    outv = (out0, out1, out2)
    sg = (sg0, sg1, sg2)
    sw = (sw0, sw1, sw2)
    so = (so0, so1, so2)

    wid = lax.axis_index("s") * 2 + lax.axis_index("c")
    start = (wid * NBLOCKS) // NW
    end = ((wid + 1) * NBLOCKS) // NW
    count = end - start

    pltpu.sync_copy(idx_hbm.at[pl.ds(start * BE, 78 * BE)], idx_v.at[pl.ds(0, 78 * BE)])

    @pl.when(count > 78)
    def _():
        pltpu.sync_copy(idx_hbm.at[pl.ds((start + 78) * BE, BE)],
                        idx_v.at[pl.ds(78 * BE, BE)])

    def kth_block(k):
        kk = jnp.minimum(k, count - 1)
        return kk, start + kk

    def issue(k, b):
        kk, block = kth_block(k)
        pltpu.async_copy(x_hbm.at[idx_v.at[pl.ds(kk * BE, BE)]], rows[b], sg[b])
        pltpu.async_copy(w_hbm.at[pl.ds(block * BE, BE)], wv[b], sw[b])

    for b in range(NBUF):
        issue(b, b)

    def step(t, _):
        for b in range(NBUF):
            k = NBUF * t + b
            kk, block = kth_block(k)
            pltpu.make_async_copy(
                x_hbm.at[idx_v.at[pl.ds(kk * BE, BE)]], rows[b], sg[b]).wait()
            pltpu.make_async_copy(
                w_hbm.at[pl.ds(block * BE, BE)], wv[b], sw[b]).wait()

            @pl.when(t >= 1)
            def _():
                pltpu.make_async_copy(
                    outv[b], out_hbm.at[pl.ds(block * BN, BN)], so[b]).wait()

            for n in range(BN):
                def jbody(j4, acc, n=n, b=b):
                    res = list(acc)
                    for u in range(4):
                        e = n * DEG + j4 * 4 + u
                        for c in range(NCHUNK):
                            ri = rows[b][e, pl.ds(c * 16, 16)]
                            ra = lax.bitcast_convert_type(ri << 16, jnp.float32)
                            rb = lax.bitcast_convert_type(ri & _HI, jnp.float32)
                            res[2 * c] = res[2 * c] + ra * wv[b][
                                e, pl.ds(c * 32, 16)]
                            res[2 * c + 1] = res[2 * c + 1] + rb * wv[b][
                                e, pl.ds(c * 32 + 16, 16)]
                    return tuple(res)

                acc = lax.fori_loop(
                    0, DEG // 4, jbody,
                    tuple(jnp.zeros((16,), jnp.float32) for _ in range(2 * NCHUNK)),
                )
                for h in range(2 * NCHUNK):
                    outv[b][n, pl.ds(h * 16, 16)] = acc[h] * (1.0 / DEG)

            @pl.when(t < KMAX // NBUF - 1)
            def _():
                issue(k + NBUF, b)

            pltpu.async_copy(outv[b], out_hbm.at[pl.ds(block * BN, BN)], so[b])
        return 0

    lax.fori_loop(0, KMAX // NBUF, step, 0)

    for b in range(NBUF):
        pltpu.make_async_copy(outv[b], out_hbm.at[pl.ds(0, BN)], so[b]).wait()


@jax.jit
def _graph_conv(x, w, idx):
    xp = x.reshape(N_NODES, NCHUNK, 2, 16).transpose(0, 1, 3, 2)
    xp = xp.reshape(N_NODES, D // 2, 2).astype(jnp.bfloat16)
    xp = jax.lax.bitcast_convert_type(xp, jnp.int32)
    mesh = plsc.VectorSubcoreMesh(core_axis_name="c", subcore_axis_name="s")
    k = functools.partial(
        pl.kernel,
        mesh=mesh,
        compiler_params=pltpu.CompilerParams(use_tc_tiling_on_sc=False),
        out_type=jax.ShapeDtypeStruct((N_NODES, D), jnp.float32),
        scratch_types=[
            pltpu.VMEM((79 * BE,), jnp.int32),
            pltpu.VMEM((BE, D // 2), jnp.int32),
            pltpu.VMEM((BE, D // 2), jnp.int32),
            pltpu.VMEM((BE, D // 2), jnp.int32),
            pltpu.VMEM((BE, D), jnp.float32),
            pltpu.VMEM((BE, D), jnp.float32),
            pltpu.VMEM((BE, D), jnp.float32),
            pltpu.VMEM((BN, D), jnp.float32),
            pltpu.VMEM((BN, D), jnp.float32),
            pltpu.VMEM((BN, D), jnp.float32),
            pltpu.SemaphoreType.DMA,
            pltpu.SemaphoreType.DMA,
            pltpu.SemaphoreType.DMA,
            pltpu.SemaphoreType.DMA,
            pltpu.SemaphoreType.DMA,
            pltpu.SemaphoreType.DMA,
            pltpu.SemaphoreType.DMA,
            pltpu.SemaphoreType.DMA,
            pltpu.SemaphoreType.DMA,
        ],
    )(_body)
    return k(xp, w, idx)


def kernel(input, weights, idxn):
    return _graph_conv(input, weights, idxn)
